# baseline (speedup 1.0000x reference)
import functools

import jax
import jax.numpy as jnp
from jax import lax
from jax.experimental import pallas as pl
from jax.experimental.pallas import tpu as pltpu
from jax.experimental.pallas import tpu_sc as plsc

NC = 2
NS = 16
NW = NC * NS
CHUNK = 128


def _mm_body(x_ref, w_ref, o_ref):
    o_ref[...] = jnp.dot(x_ref[...], w_ref[...],
                         preferred_element_type=jnp.float32)


def _matmul(x, w):
    n, _ = x.shape
    _, dout = w.shape
    return pl.pallas_call(
        _mm_body,
        out_shape=jax.ShapeDtypeStruct((n, dout), jnp.float32),
    )(x, w)


def _combine_relu_mm_body(n, p_ref, b_ref, w_ref, o_ref):
    h = p_ref[0, :n] + p_ref[1, :n] + b_ref[...]
    h = jnp.maximum(h, 0.0)
    o_ref[...] = jnp.dot(h, w_ref[...], preferred_element_type=jnp.float32)


def _combine_relu_mm(parts, b, w, n):
    dout = w.shape[1]
    return pl.pallas_call(
        functools.partial(_combine_relu_mm_body, n),
        out_shape=jax.ShapeDtypeStruct((n, dout), jnp.float32),
    )(parts, b, w)


def _combine_bias_body(n, p_ref, b_ref, o_ref):
    o_ref[...] = p_ref[0, :n] + p_ref[1, :n] + b_ref[...]


def _combine_bias(parts, b, n):
    _, _, d = parts.shape
    return pl.pallas_call(
        functools.partial(_combine_bias_body, n),
        out_shape=jax.ShapeDtypeStruct((n, d), jnp.float32),
    )(parts, b)


def _make_aggregate(n_pad, d, chunks):
    mesh = plsc.VectorSubcoreMesh(core_axis_name="c", subcore_axis_name="s")
    rows_per_sub = n_pad // NS
    n_half = chunks // 2

    @functools.partial(
        pl.kernel,
        mesh=mesh,
        out_type=jax.ShapeDtypeStruct((NC, n_pad, d), jnp.float32),
        scratch_types=[
            pltpu.VMEM((n_half, CHUNK), jnp.int32),
            pltpu.VMEM((n_half, CHUNK), jnp.int32),
            pltpu.VMEM((CHUNK, d), jnp.float32),
            pltpu.VMEM((CHUNK, d), jnp.float32),
            pltpu.VMEM_SHARED((n_pad, d), jnp.float32),
            pltpu.SemaphoreType.DMA,
            pltpu.SemaphoreType.DMA,
        ],
    )
    def aggregate(h_hbm, src_hbm, dst_hbm, zeros_hbm, out_hbm,
                  src_v, dst_v, rows_a, rows_b, acc, sem_a, sem_b):
        c = lax.axis_index("c")
        s = lax.axis_index("s")
        wid = c * NS + s
        row0 = s * rows_per_sub
        pltpu.sync_copy(zeros_hbm.at[pl.ds(row0, rows_per_sub)],
                        acc.at[pl.ds(row0, rows_per_sub)])
        plsc.subcore_barrier()

        for half in range(2):
            pltpu.sync_copy(src_hbm.at[wid, pl.ds(half * n_half, n_half)],
                            src_v)
            pltpu.sync_copy(dst_hbm.at[wid, pl.ds(half * n_half, n_half)],
                            dst_v)
            pltpu.async_copy(h_hbm.at[src_v.at[0]], rows_a, sem_a)

            def body(i, carry):
                j0 = 2 * i
                j1 = j0 + 1
                j2 = jnp.minimum(j0 + 2, n_half - 1)
                pltpu.make_async_copy(h_hbm.at[src_v.at[j0]], rows_a,
                                      sem_a).wait()
                pltpu.async_copy(h_hbm.at[src_v.at[j1]], rows_b, sem_b)
                pltpu.sync_copy(rows_a, acc.at[dst_v.at[j0]], add=True)
                pltpu.make_async_copy(h_hbm.at[src_v.at[j1]], rows_b,
                                      sem_b).wait()
                pltpu.async_copy(h_hbm.at[src_v.at[j2]], rows_a, sem_a)
                pltpu.sync_copy(rows_b, acc.at[dst_v.at[j1]], add=True)
                return carry

            lax.fori_loop(0, n_half // 2, body, 0)
            pltpu.make_async_copy(h_hbm.at[src_v.at[n_half - 1]], rows_a,
                                  sem_a).wait()
        plsc.subcore_barrier()
        pltpu.sync_copy(acc.at[pl.ds(row0, rows_per_sub)],
                        out_hbm.at[c, pl.ds(row0, rows_per_sub)])

    return aggregate


def kernel(x, edge_index, W1, b1, W2, b2):
    n, d = x.shape
    e = edge_index.shape[1]

    chunks = -(-(-(-e // (NW * CHUNK))) // 16) * 16
    e_pad = NW * chunks * CHUNK
    n_pad = -(-(n + 1) // (NS * 8)) * (NS * 8)
    src = jnp.concatenate(
        [edge_index[0], jnp.zeros((e_pad - e,), jnp.int32)]).reshape(
            NW, chunks, CHUNK)
    dst = jnp.concatenate(
        [edge_index[1], jnp.full((e_pad - e,), n, jnp.int32)]).reshape(
            NW, chunks, CHUNK)
    zeros = jnp.zeros((n_pad, d), jnp.float32)

    aggregate = _make_aggregate(n_pad, d, chunks)

    h1 = _matmul(x, W1)
    p1 = aggregate(h1, src, dst, zeros)
    h2 = _combine_relu_mm(p1, b1, W2, n)
    p2 = aggregate(h2, src, dst, zeros)
    return _combine_bias(p2, b2, n)


# speedup vs baseline: 1.4102x; 1.4102x over previous
import functools

import jax
import jax.numpy as jnp
from jax import lax
from jax.experimental import pallas as pl
from jax.experimental.pallas import tpu as pltpu
from jax.experimental.pallas import tpu_sc as plsc

NC = 2
NS = 16
NW = NC * NS
CHUNK = 128


def _mm_body(x_ref, w_ref, o_ref):
    o_ref[...] = jnp.dot(x_ref[...], w_ref[...],
                         preferred_element_type=jnp.float32)


def _matmul(x, w):
    n, _ = x.shape
    _, dout = w.shape
    return pl.pallas_call(
        _mm_body,
        out_shape=jax.ShapeDtypeStruct((n, dout), jnp.float32),
    )(x, w)


def _combine_relu_mm_body(n, p_ref, b_ref, w_ref, o_ref):
    h = p_ref[0, :n] + p_ref[1, :n] + b_ref[...]
    h = jnp.maximum(h, 0.0)
    o_ref[...] = jnp.dot(h, w_ref[...], preferred_element_type=jnp.float32)


def _combine_relu_mm(parts, b, w, n):
    dout = w.shape[1]
    return pl.pallas_call(
        functools.partial(_combine_relu_mm_body, n),
        out_shape=jax.ShapeDtypeStruct((n, dout), jnp.float32),
    )(parts, b, w)


def _combine_bias_body(n, p_ref, b_ref, o_ref):
    o_ref[...] = p_ref[0, :n] + p_ref[1, :n] + b_ref[...]


def _combine_bias(parts, b, n):
    _, _, d = parts.shape
    return pl.pallas_call(
        functools.partial(_combine_bias_body, n),
        out_shape=jax.ShapeDtypeStruct((n, d), jnp.float32),
    )(parts, b)


def _make_aggregate(n_pad, d, chunks):
    mesh = plsc.VectorSubcoreMesh(core_axis_name="c", subcore_axis_name="s")
    rows_per_sub = n_pad // NS

    @functools.partial(
        pl.kernel,
        mesh=mesh,
        out_type=jax.ShapeDtypeStruct((NC, n_pad, d), jnp.float32),
        scratch_types=[
            pltpu.VMEM((chunks, CHUNK), jnp.int32),
            pltpu.VMEM((chunks, CHUNK), jnp.int32),
            pltpu.VMEM((CHUNK, d), jnp.float32),
            pltpu.VMEM_SHARED((n_pad, d), jnp.float32),
            pltpu.SemaphoreType.DMA,
        ],
    )
    def aggregate(h_hbm, src_hbm, dst_hbm, zeros_hbm, out_hbm,
                  src_v, dst_v, rows_a, acc, sem_a):
        c = lax.axis_index("c")
        s = lax.axis_index("s")
        wid = c * NS + s
        row0 = s * rows_per_sub
        pltpu.sync_copy(zeros_hbm.at[pl.ds(row0, rows_per_sub)],
                        acc.at[pl.ds(row0, rows_per_sub)])
        pltpu.sync_copy(src_hbm.at[wid], src_v)
        pltpu.sync_copy(dst_hbm.at[wid], dst_v)
        plsc.subcore_barrier()

        def body(j, carry):
            pltpu.async_copy(h_hbm.at[src_v.at[j]], rows_a, sem_a).wait()
            pltpu.sync_copy(rows_a, acc.at[dst_v.at[j]], add=True)
            return carry

        lax.fori_loop(0, chunks, body, 0)
        plsc.subcore_barrier()
        pltpu.sync_copy(acc.at[pl.ds(row0, rows_per_sub)],
                        out_hbm.at[c, pl.ds(row0, rows_per_sub)])

    return aggregate


def kernel(x, edge_index, W1, b1, W2, b2):
    n, d = x.shape
    e = edge_index.shape[1]

    chunks = -(-e // (NW * CHUNK))
    e_pad = NW * chunks * CHUNK
    n_pad = -(-(n + 1) // (NS * 8)) * (NS * 8)
    src = jnp.concatenate(
        [edge_index[0], jnp.zeros((e_pad - e,), jnp.int32)]).reshape(
            NW, chunks, CHUNK)
    dst = jnp.concatenate(
        [edge_index[1], jnp.full((e_pad - e,), n, jnp.int32)]).reshape(
            NW, chunks, CHUNK)
    zeros = jnp.zeros((n_pad, d), jnp.float32)

    aggregate = _make_aggregate(n_pad, d, chunks)

    h1 = _matmul(x, W1)
    p1 = aggregate(h1, src, dst, zeros)
    h2 = _combine_relu_mm(p1, b1, W2, n)
    p2 = aggregate(h2, src, dst, zeros)
    return _combine_bias(p2, b2, n)


# speedup vs baseline: 1.9284x; 1.3674x over previous
import functools

import jax
import jax.numpy as jnp
from jax import lax
from jax.experimental import pallas as pl
from jax.experimental.pallas import tpu as pltpu
from jax.experimental.pallas import tpu_sc as plsc

NC = 2
NS = 16
CHUNK = 128


def _mm_split_body(n, dh, x_ref, w_ref, o_ref):
    r = jnp.dot(x_ref[...], w_ref[...], preferred_element_type=jnp.float32)
    o_ref[0, :n] = r[:, :dh]
    o_ref[1, :n] = r[:, dh:]


def _matmul_split(x, w, n_pad):
    n = x.shape[0]
    dout = w.shape[1]
    dh = dout // 2
    return pl.pallas_call(
        functools.partial(_mm_split_body, n, dh),
        out_shape=jax.ShapeDtypeStruct((2, n_pad, dh), jnp.float32),
    )(x, w)


def _combine_relu_mm_body(n, dh, p_ref, b_ref, w_ref, o_ref):
    h = jnp.concatenate([p_ref[0, :n], p_ref[1, :n]], axis=1) + b_ref[...]
    h = jnp.maximum(h, 0.0)
    r = jnp.dot(h, w_ref[...], preferred_element_type=jnp.float32)
    o_ref[0, :n] = r[:, :dh]
    o_ref[1, :n] = r[:, dh:]


def _combine_relu_mm(parts, b, w, n, n_pad):
    dout = w.shape[1]
    dh = dout // 2
    return pl.pallas_call(
        functools.partial(_combine_relu_mm_body, n, dh),
        out_shape=jax.ShapeDtypeStruct((2, n_pad, dh), jnp.float32),
    )(parts, b, w)


def _combine_bias_body(n, p_ref, b_ref, o_ref):
    o_ref[...] = (jnp.concatenate([p_ref[0, :n], p_ref[1, :n]], axis=1)
                  + b_ref[...])


def _combine_bias(parts, b, n):
    d = 2 * parts.shape[2]
    return pl.pallas_call(
        functools.partial(_combine_bias_body, n),
        out_shape=jax.ShapeDtypeStruct((n, d), jnp.float32),
    )(parts, b)


def _make_aggregate(n, n_pad, dh, chunks):
    mesh = plsc.VectorSubcoreMesh(core_axis_name="c", subcore_axis_name="s")
    rows_per_sub = n_pad // NS

    @functools.partial(
        pl.kernel,
        mesh=mesh,
        compiler_params=pltpu.CompilerParams(use_tc_tiling_on_sc=False),
        out_type=jax.ShapeDtypeStruct((NC, n_pad, dh), jnp.float32),
        scratch_types=[
            pltpu.VMEM((chunks // 4, CHUNK), jnp.int32),
            pltpu.VMEM((chunks // 4, CHUNK), jnp.int32),
            pltpu.VMEM((CHUNK, dh), jnp.float32),
            pltpu.VMEM_SHARED((2 * n_pad, dh), jnp.float32),
            pltpu.SemaphoreType.DMA,
        ],
    )
    def aggregate(h_hbm, src_hbm, dst_hbm, zeros_hbm, out_hbm,
                  src_v, dst_v, rows_v, sp, sem):
        c = lax.axis_index("c")
        s = lax.axis_index("s")
        row0 = s * rows_per_sub
        pltpu.sync_copy(h_hbm.at[c, pl.ds(row0, rows_per_sub)],
                        sp.at[pl.ds(row0, rows_per_sub)])
        pltpu.sync_copy(zeros_hbm.at[pl.ds(row0, rows_per_sub)],
                        sp.at[pl.ds(n_pad + row0, rows_per_sub)])

        plsc.subcore_barrier()
        cq = chunks // 4

        def body(j, carry):
            pltpu.async_copy(sp.at[src_v.at[j]], rows_v, sem).wait()
            pltpu.sync_copy(rows_v, sp.at[dst_v.at[j]], add=True)
            return carry

        for q in range(4):
            pltpu.sync_copy(src_hbm.at[s, pl.ds(q * cq, cq)], src_v)
            pltpu.sync_copy(dst_hbm.at[s, pl.ds(q * cq, cq)], dst_v)
            lax.fori_loop(0, cq, body, 0)
        plsc.subcore_barrier()
        pltpu.sync_copy(sp.at[pl.ds(n_pad + row0, rows_per_sub)],
                        out_hbm.at[c, pl.ds(row0, rows_per_sub)])

    return aggregate


def kernel(x, edge_index, W1, b1, W2, b2):
    n, d = x.shape
    dh = d // 2
    e = edge_index.shape[1]

    chunks = -(-(-(-e // (NS * CHUNK))) // 32) * 32
    e_pad = NS * chunks * CHUNK
    n_pad = -(-(n + 1) // (NS * 8)) * (NS * 8)
    src = jnp.concatenate(
        [edge_index[0], jnp.zeros((e_pad - e,), jnp.int32)]).reshape(
            NS, chunks, CHUNK)
    dst = jnp.concatenate(
        [edge_index[1] + n_pad, jnp.full((e_pad - e,), n_pad + n,
                                         jnp.int32)]).reshape(
            NS, chunks, CHUNK)
    zeros = jnp.zeros((n_pad, dh), jnp.float32)

    aggregate = _make_aggregate(n, n_pad, dh, chunks)

    h1 = _matmul_split(x, W1, n_pad)
    p1 = aggregate(h1, src, dst, zeros)
    h2 = _combine_relu_mm(p1, b1, W2, n, n_pad)
    p2 = aggregate(h2, src, dst, zeros)
    return _combine_bias(p2, b2, n)


# speedup vs baseline: 2.5134x; 1.3034x over previous
import functools

import jax
import jax.numpy as jnp
from jax import lax
from jax.experimental import pallas as pl
from jax.experimental.pallas import tpu as pltpu
from jax.experimental.pallas import tpu_sc as plsc

NC = 2
NS = 16
CHUNK = 80


def _mm_split_body(n, dh, x_ref, w_ref, o_ref):
    r = jnp.dot(x_ref[...], w_ref[...], preferred_element_type=jnp.float32)
    o_ref[0, :n] = r[:, :dh]
    o_ref[1, :n] = r[:, dh:]


def _matmul_split(x, w, n_pad):
    n = x.shape[0]
    dout = w.shape[1]
    dh = dout // 2
    return pl.pallas_call(
        functools.partial(_mm_split_body, n, dh),
        out_shape=jax.ShapeDtypeStruct((2, n_pad, dh), jnp.float32),
    )(x, w)


def _combine_relu_mm_body(n, dh, p_ref, b_ref, w_ref, o_ref):
    h = jnp.concatenate([p_ref[0, :n], p_ref[1, :n]], axis=1) + b_ref[...]
    h = jnp.maximum(h, 0.0)
    r = jnp.dot(h, w_ref[...], preferred_element_type=jnp.float32)
    o_ref[0, :n] = r[:, :dh]
    o_ref[1, :n] = r[:, dh:]


def _combine_relu_mm(parts, b, w, n, n_pad):
    dout = w.shape[1]
    dh = dout // 2
    return pl.pallas_call(
        functools.partial(_combine_relu_mm_body, n, dh),
        out_shape=jax.ShapeDtypeStruct((2, n_pad, dh), jnp.float32),
    )(parts, b, w)


def _combine_bias_body(n, p_ref, b_ref, o_ref):
    o_ref[...] = (jnp.concatenate([p_ref[0, :n], p_ref[1, :n]], axis=1)
                  + b_ref[...])


def _combine_bias(parts, b, n):
    d = 2 * parts.shape[2]
    return pl.pallas_call(
        functools.partial(_combine_bias_body, n),
        out_shape=jax.ShapeDtypeStruct((n, d), jnp.float32),
    )(parts, b)


def _make_aggregate(n, n_pad, dh, chunks):
    mesh = plsc.VectorSubcoreMesh(core_axis_name="c", subcore_axis_name="s")
    rows_per_sub = n_pad // NS

    @functools.partial(
        pl.kernel,
        mesh=mesh,
        compiler_params=pltpu.CompilerParams(use_tc_tiling_on_sc=False),
        out_type=jax.ShapeDtypeStruct((NC, n_pad, dh), jnp.float32),
        scratch_types=[
            pltpu.VMEM((chunks // 4, CHUNK), jnp.int32),
            pltpu.VMEM((chunks // 4, CHUNK), jnp.int32),
            pltpu.VMEM((CHUNK, dh), jnp.float32),
            pltpu.VMEM((CHUNK, dh), jnp.float32),
            pltpu.VMEM_SHARED((2 * n_pad, dh), jnp.float32),
            pltpu.SemaphoreType.DMA,
            pltpu.SemaphoreType.DMA,
            pltpu.SemaphoreType.DMA,
            pltpu.SemaphoreType.DMA,
        ],
    )
    def aggregate(h_hbm, src_hbm, dst_hbm, zeros_hbm, out_hbm,
                  src_v, dst_v, rows_a, rows_b, sp, g_a, g_b, s_a, s_b):
        c = lax.axis_index("c")
        s = lax.axis_index("s")
        row0 = s * rows_per_sub
        pltpu.sync_copy(h_hbm.at[c, pl.ds(row0, rows_per_sub)],
                        sp.at[pl.ds(row0, rows_per_sub)])
        pltpu.sync_copy(zeros_hbm.at[pl.ds(row0, rows_per_sub)],
                        sp.at[pl.ds(n_pad + row0, rows_per_sub)])

        plsc.subcore_barrier()
        cq = chunks // 4

        def body(i, carry):
            j0 = 2 * i
            j1 = j0 + 1
            pltpu.make_async_copy(sp.at[src_v.at[j0]], rows_a, g_a).wait()
            pltpu.async_copy(rows_a, sp.at[dst_v.at[j0]], s_a, add=True)
            pltpu.make_async_copy(rows_a, sp.at[dst_v.at[j0]], s_a).wait()
            pltpu.async_copy(sp.at[src_v.at[jnp.minimum(j0 + 2, cq - 1)]],
                             rows_a, g_a)
            pltpu.make_async_copy(sp.at[src_v.at[j1]], rows_b, g_b).wait()
            pltpu.async_copy(rows_b, sp.at[dst_v.at[j1]], s_b, add=True)
            pltpu.make_async_copy(rows_b, sp.at[dst_v.at[j1]], s_b).wait()
            pltpu.async_copy(sp.at[src_v.at[jnp.minimum(j1 + 2, cq - 1)]],
                             rows_b, g_b)
            return carry

        for q in range(4):
            pltpu.sync_copy(src_hbm.at[s, pl.ds(q * cq, cq)], src_v)
            pltpu.sync_copy(dst_hbm.at[s, pl.ds(q * cq, cq)], dst_v)
            pltpu.async_copy(sp.at[src_v.at[0]], rows_a, g_a)
            pltpu.async_copy(sp.at[src_v.at[1]], rows_b, g_b)
            lax.fori_loop(0, cq // 2, body, 0)
            pltpu.make_async_copy(sp.at[src_v.at[cq - 1]], rows_a, g_a).wait()
            pltpu.make_async_copy(sp.at[src_v.at[cq - 1]], rows_b, g_b).wait()
        plsc.subcore_barrier()
        pltpu.sync_copy(sp.at[pl.ds(n_pad + row0, rows_per_sub)],
                        out_hbm.at[c, pl.ds(row0, rows_per_sub)])

    return aggregate


def kernel(x, edge_index, W1, b1, W2, b2):
    n, d = x.shape
    dh = d // 2
    e = edge_index.shape[1]

    chunks = -(-(-(-e // (NS * CHUNK))) // 32) * 32
    e_pad = NS * chunks * CHUNK
    n_pad = -(-(n + 1) // (NS * 8)) * (NS * 8)
    src = jnp.concatenate(
        [edge_index[0], jnp.zeros((e_pad - e,), jnp.int32)]).reshape(
            NS, chunks, CHUNK)
    dst = jnp.concatenate(
        [edge_index[1] + n_pad, jnp.full((e_pad - e,), n_pad + n,
                                         jnp.int32)]).reshape(
            NS, chunks, CHUNK)
    zeros = jnp.zeros((n_pad, dh), jnp.float32)

    aggregate = _make_aggregate(n, n_pad, dh, chunks)

    h1 = _matmul_split(x, W1, n_pad)
    p1 = aggregate(h1, src, dst, zeros)
    h2 = _combine_relu_mm(p1, b1, W2, n, n_pad)
    p2 = aggregate(h2, src, dst, zeros)
    return _combine_bias(p2, b2, n)


# speedup vs baseline: 2.5784x; 1.0259x over previous
import functools

import jax
import jax.numpy as jnp
from jax import lax
from jax.experimental import pallas as pl
from jax.experimental.pallas import tpu as pltpu
from jax.experimental.pallas import tpu_sc as plsc

NC = 2
NS = 16
CHUNK = 80


def _mm_split_body(n, dh, x_ref, w_ref, o_ref):
    r = jnp.dot(x_ref[...], w_ref[...], preferred_element_type=jnp.float32)
    o_ref[0, :n] = r[:, :dh]
    o_ref[1, :n] = r[:, dh:]


def _matmul_split(x, w, n_pad):
    n = x.shape[0]
    dout = w.shape[1]
    dh = dout // 2
    return pl.pallas_call(
        functools.partial(_mm_split_body, n, dh),
        out_shape=jax.ShapeDtypeStruct((2, n_pad, dh), jnp.float32),
    )(x, w)


def _relu_mm_body(n, dh, p_ref, w_ref, o_ref):
    h = jnp.maximum(p_ref[:n], 0.0)
    r = jnp.dot(h, w_ref[...], preferred_element_type=jnp.float32)
    o_ref[0, :n] = r[:, :dh]
    o_ref[1, :n] = r[:, dh:]


def _relu_mm(parts, w, n, n_pad):
    dout = w.shape[1]
    dh = dout // 2
    return pl.pallas_call(
        functools.partial(_relu_mm_body, n, dh),
        out_shape=jax.ShapeDtypeStruct((2, n_pad, dh), jnp.float32),
    )(parts, w)


def _make_aggregate(n, n_pad, dh, chunks):
    mesh = plsc.VectorSubcoreMesh(core_axis_name="c", subcore_axis_name="s")
    rows_per_sub = n_pad // NS

    @functools.partial(
        pl.kernel,
        mesh=mesh,
        compiler_params=pltpu.CompilerParams(use_tc_tiling_on_sc=False),
        out_type=jax.ShapeDtypeStruct((n_pad, 2 * dh), jnp.float32),
        scratch_types=[
            pltpu.VMEM((chunks // 4, CHUNK), jnp.int32),
            pltpu.VMEM((chunks // 4, CHUNK), jnp.int32),
            pltpu.VMEM((CHUNK, dh), jnp.float32),
            pltpu.VMEM((CHUNK, dh), jnp.float32),
            pltpu.VMEM_SHARED((2 * n_pad, dh), jnp.float32),
            pltpu.SemaphoreType.DMA,
            pltpu.SemaphoreType.DMA,
            pltpu.SemaphoreType.DMA,
            pltpu.SemaphoreType.DMA,
        ],
    )
    def aggregate(h_hbm, src_hbm, dst_hbm, init_hbm, out_hbm,
                  src_v, dst_v, rows_a, rows_b, sp, g_a, g_b, s_a, s_b):
        c = lax.axis_index("c")
        s = lax.axis_index("s")
        row0 = s * rows_per_sub
        pltpu.sync_copy(h_hbm.at[c, pl.ds(row0, rows_per_sub)],
                        sp.at[pl.ds(row0, rows_per_sub)])
        pltpu.sync_copy(init_hbm.at[c, pl.ds(row0, rows_per_sub)],
                        sp.at[pl.ds(n_pad + row0, rows_per_sub)])

        plsc.subcore_barrier()
        cq = chunks // 4

        def body(i, carry):
            j0 = 2 * i
            j1 = j0 + 1
            pltpu.make_async_copy(sp.at[src_v.at[j0]], rows_a, g_a).wait()
            pltpu.async_copy(rows_a, sp.at[dst_v.at[j0]], s_a, add=True)
            pltpu.make_async_copy(rows_a, sp.at[dst_v.at[j0]], s_a).wait()
            pltpu.async_copy(sp.at[src_v.at[jnp.minimum(j0 + 2, cq - 1)]],
                             rows_a, g_a)
            pltpu.make_async_copy(sp.at[src_v.at[j1]], rows_b, g_b).wait()
            pltpu.async_copy(rows_b, sp.at[dst_v.at[j1]], s_b, add=True)
            pltpu.make_async_copy(rows_b, sp.at[dst_v.at[j1]], s_b).wait()
            pltpu.async_copy(sp.at[src_v.at[jnp.minimum(j1 + 2, cq - 1)]],
                             rows_b, g_b)
            return carry

        for q in range(4):
            pltpu.sync_copy(src_hbm.at[s, pl.ds(q * cq, cq)], src_v)
            pltpu.sync_copy(dst_hbm.at[s, pl.ds(q * cq, cq)], dst_v)
            pltpu.async_copy(sp.at[src_v.at[0]], rows_a, g_a)
            pltpu.async_copy(sp.at[src_v.at[1]], rows_b, g_b)
            lax.fori_loop(0, cq // 2, body, 0)
            pltpu.make_async_copy(sp.at[src_v.at[cq - 1]], rows_a, g_a).wait()
            pltpu.make_async_copy(sp.at[src_v.at[cq - 1]], rows_b, g_b).wait()
        plsc.subcore_barrier()
        pltpu.sync_copy(sp.at[pl.ds(n_pad + row0, rows_per_sub)],
                        out_hbm.at[pl.ds(row0, rows_per_sub),
                                   pl.ds(c * dh, dh)])

    return aggregate


def kernel(x, edge_index, W1, b1, W2, b2):
    n, d = x.shape
    dh = d // 2
    e = edge_index.shape[1]

    chunks = -(-(-(-e // (NS * CHUNK))) // 32) * 32
    e_pad = NS * chunks * CHUNK
    n_pad = -(-(n + 1) // (NS * 8)) * (NS * 8)
    src = jnp.concatenate(
        [edge_index[0], jnp.zeros((e_pad - e,), jnp.int32)]).reshape(
            NS, chunks, CHUNK)
    dst = jnp.concatenate(
        [edge_index[1] + n_pad, jnp.full((e_pad - e,), n_pad + n,
                                         jnp.int32)]).reshape(
            NS, chunks, CHUNK)
    init1 = jnp.broadcast_to(b1.reshape(2, 1, dh), (2, n_pad, dh))
    init2 = jnp.broadcast_to(b2.reshape(2, 1, dh), (2, n_pad, dh))

    aggregate = _make_aggregate(n, n_pad, dh, chunks)

    h1 = _matmul_split(x, W1, n_pad)
    p1 = aggregate(h1, src, dst, init1)
    h2 = _relu_mm(p1, W2, n, n_pad)
    p2 = aggregate(h2, src, dst, init2)
    return p2[:n]
